# Initial kernel scaffold; baseline (speedup 1.0000x reference)
import jax
import jax.numpy as jnp
from jax.experimental import pallas as pl


def kernel(x, edge_index):
    raise NotImplementedError("write your pallas kernel here")


# speedup vs baseline: 1.6167x; 1.6167x over previous
import functools

import jax
import jax.numpy as jnp
from jax import lax
from jax.experimental import pallas as pl
from jax.experimental.pallas import tpu as pltpu
from jax.experimental.pallas import tpu_sc as plsc

D = 128
L = 16
CH = 128
NW = 32
E = 320000
CHUNKS = -(-E // (CH * NW))
EPW = CH * CHUNKS
EP = EPW * NW


def _lane_perm(v, idx):
    dnums = lax.GatherDimensionNumbers(
        offset_dims=(), collapsed_slice_dims=(0,), start_index_map=(0,))
    return lax.gather(v, idx[:, None], dnums, (1,),
                      mode=lax.GatherScatterMode.PROMISE_IN_BOUNDS)


def _make_ip_kernel():
    mesh = plsc.VectorSubcoreMesh(core_axis_name="c", subcore_axis_name="s")

    @functools.partial(
        pl.kernel,
        mesh=mesh,
        out_type=jax.ShapeDtypeStruct((EP,), jnp.float32),
        scratch_types=[
            pltpu.VMEM((CH,), jnp.int32),
            pltpu.VMEM((CH,), jnp.int32),
            pltpu.VMEM((CH, D), jnp.float32),
            pltpu.VMEM((CH, D), jnp.float32),
            pltpu.VMEM((CH,), jnp.float32),
            pltpu.SemaphoreType.DMA,
            pltpu.SemaphoreType.DMA,
        ],
    )
    def ip(x_hbm, src_hbm, dst_hbm, out_hbm,
           idx_s, idx_d, rows_s, rows_d, out_v, sem_s, sem_d):
        wid = lax.axis_index("s") * 2 + lax.axis_index("c")
        base = wid * EPW
        lane = lax.iota(jnp.int32, L)

        def chunk_body(c, carry):
            off = base + c * CH
            pltpu.sync_copy(src_hbm.at[pl.ds(off, CH)], idx_s)
            pltpu.sync_copy(dst_hbm.at[pl.ds(off, CH)], idx_d)
            cp_s = pltpu.async_copy(x_hbm.at[idx_s], rows_s, sem_s)
            cp_d = pltpu.async_copy(x_hbm.at[idx_d], rows_d, sem_d)
            cp_s.wait()
            cp_d.wait()

            def group_body(g, _):
                e0 = g * L
                res = jnp.zeros((L,), jnp.float32)
                for j in range(L):
                    e = e0 + j
                    acc = rows_s[e, pl.ds(0, L)] * rows_d[e, pl.ds(0, L)]
                    for k in range(1, D // L):
                        acc = acc + (rows_s[e, pl.ds(k * L, L)]
                                     * rows_d[e, pl.ds(k * L, L)])
                    for s in (8, 4, 2, 1):
                        acc = acc + _lane_perm(acc, lane ^ s)
                    res = jnp.where(lane == j, acc, res)
                out_v[pl.ds(e0, L)] = res
                return 0

            lax.fori_loop(0, CH // L, group_body, 0)
            pltpu.sync_copy(out_v, out_hbm.at[pl.ds(off, CH)])
            return carry

        lax.fori_loop(0, CHUNKS, chunk_body, 0)

    return ip


_ip_kernel = _make_ip_kernel()


@jax.jit
def kernel(x, edge_index):
    ei = edge_index.astype(jnp.int32)
    src = jnp.pad(ei[0], (0, EP - E))
    dst = jnp.pad(ei[1], (0, EP - E))
    out = _ip_kernel(x, src, dst)
    return out[:E]
